# baseline (speedup 1.0000x reference)
import functools

import jax
import jax.numpy as jnp
from jax.experimental import pallas as pl
from jax.experimental.pallas import tpu as pltpu


def _se_kernel(x_ref, ones_ref, w1_ref, b1_ref, w2_ref, b2_ref, o_ref, *, inv_t):
    xb = x_ref[0]
    s = jnp.dot(xb, ones_ref[...], preferred_element_type=jnp.float32)
    mean = s * jnp.float32(inv_t)
    h = jnp.dot(w1_ref[...], mean, preferred_element_type=jnp.float32)
    h = jnp.maximum(h + b1_ref[...], 0.0)
    g = jnp.dot(w2_ref[...], h, preferred_element_type=jnp.float32)
    gate = jax.nn.sigmoid(g + b2_ref[...])
    o_ref[0] = xb * gate


def kernel(x, w1, b1, w2, b2):
    B, C, T = x.shape
    H = w1.shape[0]
    ones_t = jnp.ones((T, 1), jnp.float32)
    b1c = jnp.asarray(b1, jnp.float32).reshape(H, 1)
    b2c = jnp.asarray(b2, jnp.float32).reshape(C, 1)
    w1f = jnp.asarray(w1, jnp.float32)
    w2f = jnp.asarray(w2, jnp.float32)

    return pl.pallas_call(
        functools.partial(_se_kernel, inv_t=1.0 / T),
        out_shape=jax.ShapeDtypeStruct((B, C, T), x.dtype),
        grid=(B,),
        in_specs=[
            pl.BlockSpec((1, C, T), lambda b: (b, 0, 0)),
            pl.BlockSpec((T, 1), lambda b: (0, 0)),
            pl.BlockSpec((H, C), lambda b: (0, 0)),
            pl.BlockSpec((H, 1), lambda b: (0, 0)),
            pl.BlockSpec((C, H), lambda b: (0, 0)),
            pl.BlockSpec((C, 1), lambda b: (0, 0)),
        ],
        out_specs=pl.BlockSpec((1, C, T), lambda b: (b, 0, 0)),
        compiler_params=pltpu.CompilerParams(
            dimension_semantics=("parallel",),
            vmem_limit_bytes=64 * 1024 * 1024,
        ),
    )(x, ones_t, w1f, b1c, w2f, b2c)


# speedup vs baseline: 1.4218x; 1.4218x over previous
import functools

import jax
import jax.numpy as jnp
from jax.experimental import pallas as pl
from jax.experimental.pallas import tpu as pltpu


def _se_kernel(x_ref, w1t_ref, b1_ref, w2t_ref, b2_ref, o_ref, *, inv_t):
    xb = x_ref[...]
    mean = jnp.sum(xb, axis=-1) * jnp.float32(inv_t)
    h = jnp.dot(mean, w1t_ref[...], preferred_element_type=jnp.float32)
    h = jnp.maximum(h + b1_ref[...], 0.0)
    s = jnp.dot(h, w2t_ref[...], preferred_element_type=jnp.float32)
    gate = jax.nn.sigmoid(s + b2_ref[...])
    o_ref[...] = xb * gate[:, :, None]


def kernel(x, w1, b1, w2, b2):
    B, C, T = x.shape
    H = w1.shape[0]
    BB = 4

    w1t = jnp.asarray(w1, jnp.float32).T
    w2t = jnp.asarray(w2, jnp.float32).T
    b1r = jnp.asarray(b1, jnp.float32).reshape(1, H)
    b2r = jnp.asarray(b2, jnp.float32).reshape(1, C)

    return pl.pallas_call(
        functools.partial(_se_kernel, inv_t=1.0 / T),
        out_shape=jax.ShapeDtypeStruct((B, C, T), x.dtype),
        grid=(B // BB,),
        in_specs=[
            pl.BlockSpec((BB, C, T), lambda b: (b, 0, 0)),
            pl.BlockSpec((C, H), lambda b: (0, 0)),
            pl.BlockSpec((1, H), lambda b: (0, 0)),
            pl.BlockSpec((H, C), lambda b: (0, 0)),
            pl.BlockSpec((1, C), lambda b: (0, 0)),
        ],
        out_specs=pl.BlockSpec((BB, C, T), lambda b: (b, 0, 0)),
        compiler_params=pltpu.CompilerParams(
            dimension_semantics=("parallel",),
            vmem_limit_bytes=64 * 1024 * 1024,
        ),
    )(x, w1t, b1r, w2t, b2r)
